# baseline (speedup 1.0000x reference)
import functools

import jax
import jax.numpy as jnp
from jax import lax
from jax.experimental import pallas as pl
from jax.experimental.pallas import tpu as pltpu
from jax.experimental.pallas import tpu_sc as plsc

B, H, W, C = 8, 224, 224, 96
N = H * W * C
ND2 = N // 2
SLAB_OUT = W * C // 2
NC, NS = 2, 16
HPW = H // 4
NBUF = 2

_mesh = plsc.VectorSubcoreMesh(core_axis_name="c", subcore_axis_name="s")


@functools.partial(
    pl.kernel,
    mesh=_mesh,
    out_type=jax.ShapeDtypeStruct((B, 2, ND2), jnp.float32),
    scratch_types=[
        pltpu.VMEM((NBUF, C, W), jnp.float32),
        pltpu.VMEM((NBUF, 2, SLAB_OUT), jnp.float32),
        pltpu.SemaphoreType.DMA,
        pltpu.SemaphoreType.DMA,
        pltpu.SemaphoreType.DMA,
        pltpu.SemaphoreType.DMA,
    ],
    compiler_params=pltpu.CompilerParams(
        needs_layout_passes=False,
        use_tc_tiling_on_sc=True,
        disable_bounds_checks=True,
    ),
)
def _deinterleave(in_hbm, out_hbm, ibuf, obuf, sin0, sin1, sout0, sout1):
    wid = lax.axis_index("s") * NC + lax.axis_index("c")
    ex = wid // 4
    q = wid % 4
    h0 = q * HPW
    sins = (sin0, sin1)
    souts = (sout0, sout1)
    lane = lax.iota(jnp.int32, 16)

    lane48 = lane * 48
    colvs = [lane + 16 * wb for wb in range(W // 16)]
    ridxs = [lane * 0, lane * 0 + 1]

    def in_copy(h, slot):
        return pltpu.make_async_copy(
            in_hbm.at[ex, h0 + h], ibuf.at[slot], sins[slot]
        )

    def out_copy(h, slot):
        dst = pl.multiple_of((h0 + h) * SLAB_OUT, 128)
        return pltpu.make_async_copy(
            obuf.at[slot], out_hbm.at[ex, :, pl.ds(dst, SLAB_OUT)], souts[slot]
        )

    in_copy(0, 0).start()
    in_copy(1, 1).start()

    def outer(k0, _):
        for slot in range(NBUF):
            h = k0 * NBUF + slot
            in_copy(h, slot).wait()

            @pl.when(k0 > 0)
            def _():
                out_copy(h, slot).wait()

            src = ibuf.at[slot]

            dst = obuf.at[slot]

            @plsc.parallel_loop(0, 48, unroll=2)
            def _loop(d):
                t = d + lane
                c2 = jnp.where(t >= 48, t - 48, t)
                jbase = lane48 + c2
                c22 = c2 * 2
                for r in range(2):
                    rowv = c22 + r if r else c22
                    for wb in range(W // 16):
                        val = plsc.load_gather(src, [rowv, colvs[wb]])
                        jv = jbase + 768 * wb
                        plsc.store_scatter(dst, [ridxs[r], jv], val)
            out_copy(h, slot).start()

            @pl.when(h + NBUF < HPW)
            def _():
                in_copy(h + NBUF, slot).start()

        return 0

    lax.fori_loop(0, HPW // NBUF, outer, 0)
    for slot in range(NBUF):
        out_copy(HPW - NBUF + slot, slot).wait()


def kernel(input_tensor):
    xt = jnp.transpose(input_tensor, (0, 1, 3, 2))
    return _deinterleave(xt)


# speedup vs baseline: 2.1864x; 2.1864x over previous
import functools

import jax
import jax.numpy as jnp
from jax import lax
from jax.experimental import pallas as pl
from jax.experimental.pallas import tpu as pltpu
from jax.experimental.pallas import tpu_sc as plsc

B, H, W, C = 8, 224, 224, 96
N = H * W * C
ND2 = N // 2
SLAB_OUT = W * C // 2
NC, NS = 2, 16
HPW = H // 4
NBUF = 2

_mesh = plsc.VectorSubcoreMesh(core_axis_name="c", subcore_axis_name="s")


@functools.partial(
    pl.kernel,
    mesh=_mesh,
    out_type=jax.ShapeDtypeStruct((B, 2, ND2), jnp.float32),
    scratch_types=[
        pltpu.VMEM((NBUF, C, W), jnp.float32),
        pltpu.VMEM((NBUF, 2, SLAB_OUT), jnp.float32),
        pltpu.SemaphoreType.DMA,
        pltpu.SemaphoreType.DMA,
        pltpu.SemaphoreType.DMA,
        pltpu.SemaphoreType.DMA,
    ],
    compiler_params=pltpu.CompilerParams(
        needs_layout_passes=False,
        use_tc_tiling_on_sc=True,
        disable_bounds_checks=True,
    ),
)
def _deinterleave(in_hbm, out_hbm, ibuf, obuf, sin0, sin1, sout0, sout1):
    wid = lax.axis_index("s") * NC + lax.axis_index("c")
    ex = wid // 4
    q = wid % 4
    h0 = q * HPW
    sins = (sin0, sin1)
    souts = (sout0, sout1)
    lane = lax.iota(jnp.int32, 16)

    lane48 = lane * 48
    colvs = [lane + 16 * wb for wb in range(W // 16)]
    ridxs = [lane * 0, lane * 0 + 1]

    def in_copy(h, slot):
        return pltpu.make_async_copy(
            in_hbm.at[ex, h0 + h], ibuf.at[slot], sins[slot]
        )

    def out_copy(h, slot):
        dst = pl.multiple_of((h0 + h) * SLAB_OUT, 128)
        return pltpu.make_async_copy(
            obuf.at[slot], out_hbm.at[ex, :, pl.ds(dst, SLAB_OUT)], souts[slot]
        )

    in_copy(0, 0).start()
    in_copy(1, 1).start()

    def outer(k0, _):
        for slot in range(NBUF):
            h = k0 * NBUF + slot
            in_copy(h, slot).wait()

            @pl.when(k0 > 0)
            def _():
                out_copy(h, slot).wait()

            src = ibuf.at[slot]

            dst = obuf.at[slot]

            @plsc.parallel_loop(0, 48)
            def _loop(d):
                t = d + lane
                c2 = jnp.where(t >= 48, t - 48, t)
                jbase = lane48 + c2
                c22 = c2 * 2
                for r in range(2):
                    rowv = c22 + r if r else c22
                    for wb in range(W // 16):
                        val = plsc.load_gather(src, [rowv, colvs[wb]])
                        jv = jbase + 768 * wb
                        plsc.store_scatter(dst, [ridxs[r], jv], val)
            out_copy(h, slot).start()

            @pl.when(h + NBUF < HPW)
            def _():
                in_copy(h + NBUF, slot).start()

        return 0

    lax.fori_loop(0, HPW // NBUF, outer, 0)
    for slot in range(NBUF):
        out_copy(HPW - NBUF + slot, slot).wait()


def kernel(input_tensor):
    xt = jnp.transpose(input_tensor, (0, 1, 3, 2))
    return _deinterleave(xt)
